# Initial kernel scaffold; baseline (speedup 1.0000x reference)
import jax
import jax.numpy as jnp
from jax.experimental import pallas as pl


def kernel(x, edge_index, W1, b1, W2, b2):
    raise NotImplementedError("write your pallas kernel here")


# speedup vs baseline: 28.5095x; 28.5095x over previous
import functools

import jax
import jax.numpy as jnp
from jax import lax
from jax.experimental import pallas as pl
from jax.experimental.pallas import tpu as pltpu
from jax.experimental.pallas import tpu_sc as plsc

N = 10000
NP = 10240
E = 160000
NC = 2
NS = 16
NW = NC * NS
EPT = E // NW
K = 40
NCHUNK = EPT // K
RB = 5
NROUND = NCHUNK // RB
RPT = NP // NS

_MESH = plsc.VectorSubcoreMesh(core_axis_name="c", subcore_axis_name="s")


def _make_deg_kernel():
  @functools.partial(
      pl.kernel,
      out_type=jax.ShapeDtypeStruct((NC, NP), jnp.float32),
      mesh=_MESH,
      compiler_params=pltpu.CompilerParams(use_tc_tiling_on_sc=False),
      scratch_types=[
          pltpu.VMEM((NCHUNK, K), jnp.int32),
          pltpu.VMEM((K,), jnp.float32),
          pltpu.VMEM_SHARED((NP,), jnp.float32),
      ],
  )
  def deg_kernel(col_hbm, ones_hbm, zeros_hbm, out_hbm, col_v, ones_v, dacc):
    c = lax.axis_index("c")
    s = lax.axis_index("s")
    t = s * NC + c
    base = s * RPT
    pltpu.sync_copy(zeros_hbm.at[pl.ds(base, RPT)], dacc.at[pl.ds(base, RPT)])
    pltpu.sync_copy(col_hbm.at[t], col_v)
    pltpu.sync_copy(ones_hbm, ones_v)
    plsc.subcore_barrier()

    def body(j, carry):
      pltpu.sync_copy(ones_v, dacc.at[col_v.at[j]], add=True)
      return carry

    lax.fori_loop(0, NCHUNK, body, 0)
    plsc.subcore_barrier()
    pltpu.sync_copy(dacc.at[pl.ds(base, RPT)],
                    out_hbm.at[c].at[pl.ds(base, RPT)])

  return deg_kernel


def _make_agg_kernel(D, nphase):
  @functools.partial(
      pl.kernel,
      out_type=jax.ShapeDtypeStruct((nphase, NC, NP, D), jnp.float32),
      mesh=_MESH,
      compiler_params=pltpu.CompilerParams(use_tc_tiling_on_sc=False),
      scratch_types=[
          pltpu.VMEM((NCHUNK, K), jnp.int32),
          pltpu.VMEM((NCHUNK, K), jnp.int32),
          pltpu.VMEM((RB * K, D), jnp.float32),
          pltpu.VMEM_SHARED((NP, D), jnp.float32),
          pltpu.SemaphoreType.DMA,
      ],
  )
  def agg_kernel(row_hbm, col_hbm, src_hbm, zeros_hbm, out_hbm,
                 row_v, col_v, buf, acc, sem):
    c = lax.axis_index("c")
    s = lax.axis_index("s")
    t = s * NC + c
    base = s * RPT

    pltpu.sync_copy(row_hbm.at[t], row_v)
    pltpu.sync_copy(col_hbm.at[t], col_v)

    for p in range(nphase):
      @pl.when(c == 0)
      def _():
        pltpu.sync_copy(src_hbm.at[p].at[pl.ds(base, RPT)],
                        acc.at[pl.ds(base, RPT)])

      @pl.when(c != 0)
      def _():
        pltpu.sync_copy(zeros_hbm.at[pl.ds(base, RPT)],
                        acc.at[pl.ds(base, RPT)])

      plsc.subcore_barrier()

      def round_body(r, carry):
        j0 = r * RB
        descs = [
            pltpu.async_copy(src_hbm.at[p].at[row_v.at[j0 + b]],
                             buf.at[pl.ds(b * K, K)], sem)
            for b in range(RB)
        ]
        for d in descs:
          d.wait()
        for b in range(RB):
          pltpu.sync_copy(buf.at[pl.ds(b * K, K)],
                          acc.at[col_v.at[j0 + b]], add=True)
        return carry

      lax.fori_loop(0, NROUND, round_body, 0)
      plsc.subcore_barrier()
      pltpu.sync_copy(acc.at[pl.ds(base, RPT)],
                      out_hbm.at[p].at[c].at[pl.ds(base, RPT)])

  return agg_kernel


_deg_kernel = _make_deg_kernel()
_agg64x2 = _make_agg_kernel(64, 2)
_agg48 = _make_agg_kernel(48, 1)


_RBLK = 640
_NBLK = NP // _RBLK


def _prescale_body(deg_ref, x_ref, xs_ref, dinv_ref):
  deg = deg_ref[:, 0:1] + deg_ref[:, 1:2] + 1.0
  dinv = lax.rsqrt(deg)
  dinv_ref[...] = dinv
  xs_ref[0] = x_ref[:, :64] * dinv
  xs_ref[1] = x_ref[:, 64:] * dinv


def _tc_prescale(deg_t, x_pad):
  return pl.pallas_call(
      _prescale_body,
      grid=(_NBLK,),
      in_specs=[
          pl.BlockSpec((_RBLK, NC), lambda i: (i, 0)),
          pl.BlockSpec((_RBLK, 128), lambda i: (i, 0)),
      ],
      out_specs=[
          pl.BlockSpec((2, _RBLK, 64), lambda i: (0, i, 0)),
          pl.BlockSpec((_RBLK, 1), lambda i: (i, 0)),
      ],
      out_shape=[
          jax.ShapeDtypeStruct((2, NP, 64), jnp.float32),
          jax.ShapeDtypeStruct((NP, 1), jnp.float32),
      ],
  )(deg_t, x_pad)


def _mm_body(p_ref, dinv_ref, w1_ref, b1_ref, w2_ref, ys_ref):
  dinv = dinv_ref[...]
  a = jnp.concatenate([p_ref[0, 0] + p_ref[0, 1],
                       p_ref[1, 0] + p_ref[1, 1]], axis=1) * dinv
  h = jnp.dot(a, w1_ref[...], preferred_element_type=jnp.float32)
  h = jnp.maximum(h + b1_ref[...], 0.0)
  y = jnp.dot(h, w2_ref[...], preferred_element_type=jnp.float32)
  ys_ref[...] = y * dinv


def _tc_mm(p, dinv, w1, b1, w2p):
  return pl.pallas_call(
      _mm_body,
      grid=(_NBLK,),
      in_specs=[
          pl.BlockSpec((2, NC, _RBLK, 64), lambda i: (0, 0, i, 0)),
          pl.BlockSpec((_RBLK, 1), lambda i: (i, 0)),
          pl.BlockSpec((128, 2048), lambda i: (0, 0)),
          pl.BlockSpec((1, 2048), lambda i: (0, 0)),
          pl.BlockSpec((2048, 48), lambda i: (0, 0)),
      ],
      out_specs=pl.BlockSpec((_RBLK, 48), lambda i: (i, 0)),
      out_shape=jax.ShapeDtypeStruct((NP, 48), jnp.float32),
  )(p, dinv, w1, b1, w2p)


def _final_body(q_ref, dinv_ref, b2_ref, out_ref):
  out_ref[...] = (q_ref[0] + q_ref[1]) * dinv_ref[...] + b2_ref[...]


def _tc_final(q, dinv, b2p):
  return pl.pallas_call(
      _final_body,
      grid=(_NBLK,),
      in_specs=[
          pl.BlockSpec((NC, _RBLK, 48), lambda i: (0, i, 0)),
          pl.BlockSpec((_RBLK, 1), lambda i: (i, 0)),
          pl.BlockSpec((1, 48), lambda i: (0, 0)),
      ],
      out_specs=pl.BlockSpec((_RBLK, 48), lambda i: (i, 0)),
      out_shape=jax.ShapeDtypeStruct((NP, 48), jnp.float32),
  )(q, dinv, b2p)


def kernel(x, edge_index, W1, b1, W2, b2):
  ei = edge_index.astype(jnp.int32)
  row2 = ei[0].reshape(NW, NCHUNK, K)
  col2 = ei[1].reshape(NW, NCHUNK, K)
  x_pad = jnp.pad(x, ((0, NP - N), (0, 0)))
  w2p = jnp.pad(W2, ((0, 0), (0, 48 - W2.shape[1])))
  b1r = b1.reshape(1, 2048)
  b2p = jnp.pad(b2, (0, 48 - b2.shape[0])).reshape(1, 48)
  ones_k = jnp.ones((K,), jnp.float32)
  z1 = jnp.zeros((NP,), jnp.float32)
  z64 = jnp.zeros((NP, 64), jnp.float32)
  z48 = jnp.zeros((NP, 48), jnp.float32)

  degp = _deg_kernel(col2, ones_k, z1)
  deg_t = degp.T
  xs2, dinv = _tc_prescale(deg_t, x_pad)
  p = _agg64x2(row2, col2, xs2, z64)
  ys = _tc_mm(p, dinv, W1, b1r, w2p)
  q = _agg48(row2, col2, ys[None], z48)
  outp = _tc_final(q[0], dinv, b2p)
  return outp[:N, :40]


# speedup vs baseline: 33.7412x; 1.1835x over previous
import functools

import jax
import jax.numpy as jnp
from jax import lax
from jax.experimental import pallas as pl
from jax.experimental.pallas import tpu as pltpu
from jax.experimental.pallas import tpu_sc as plsc

N = 10000
NP = 10240
E = 160000
NC = 2
NS = 16
NW = NC * NS
EPT = E // NW
K = 40
NCHUNK = EPT // K
RB = 5
NROUND = NCHUNK // RB
RPT = NP // NS

_MESH = plsc.VectorSubcoreMesh(core_axis_name="c", subcore_axis_name="s")


def _make_deg_kernel():
  @functools.partial(
      pl.kernel,
      out_type=jax.ShapeDtypeStruct((NC, NP), jnp.float32),
      mesh=_MESH,
      compiler_params=pltpu.CompilerParams(use_tc_tiling_on_sc=False),
      scratch_types=[
          pltpu.VMEM((NCHUNK, K), jnp.int32),
          pltpu.VMEM((K,), jnp.float32),
          pltpu.VMEM_SHARED((NP,), jnp.float32),
      ],
  )
  def deg_kernel(col_hbm, ones_hbm, zeros_hbm, out_hbm, col_v, ones_v, dacc):
    c = lax.axis_index("c")
    s = lax.axis_index("s")
    t = s * NC + c
    base = s * RPT
    pltpu.sync_copy(zeros_hbm.at[pl.ds(base, RPT)], dacc.at[pl.ds(base, RPT)])
    pltpu.sync_copy(col_hbm.at[t], col_v)
    pltpu.sync_copy(ones_hbm, ones_v)
    plsc.subcore_barrier()

    def body(j, carry):
      pltpu.sync_copy(ones_v, dacc.at[col_v.at[j]], add=True)
      return carry

    lax.fori_loop(0, NCHUNK, body, 0)
    plsc.subcore_barrier()
    pltpu.sync_copy(dacc.at[pl.ds(base, RPT)],
                    out_hbm.at[c].at[pl.ds(base, RPT)])

  return deg_kernel


def _make_agg_kernel(D, nphase):
  CH = RB * K

  @functools.partial(
      pl.kernel,
      out_type=jax.ShapeDtypeStruct((nphase, NC, NP, D), jnp.float32),
      mesh=_MESH,
      compiler_params=pltpu.CompilerParams(use_tc_tiling_on_sc=False),
      scratch_types=[
          pltpu.VMEM((NCHUNK, K), jnp.int32),
          pltpu.VMEM((NCHUNK, K), jnp.int32),
          pltpu.VMEM((2 * CH, D), jnp.float32),
          pltpu.VMEM_SHARED((NP, D), jnp.float32),
          pltpu.SemaphoreType.DMA,
          pltpu.SemaphoreType.DMA,
      ],
  )
  def agg_kernel(row_hbm, col_hbm, src_hbm, zeros_hbm, out_hbm,
                 row_v, col_v, buf, acc, gsem, ssem):
    c = lax.axis_index("c")
    s = lax.axis_index("s")
    t = s * NC + c
    base = s * RPT

    pltpu.sync_copy(row_hbm.at[t], row_v)
    pltpu.sync_copy(col_hbm.at[t], col_v)

    for p in range(nphase):
      @pl.when(c == 0)
      def _():
        pltpu.sync_copy(src_hbm.at[p].at[pl.ds(base, RPT)],
                        acc.at[pl.ds(base, RPT)])

      @pl.when(c != 0)
      def _():
        pltpu.sync_copy(zeros_hbm.at[pl.ds(base, RPT)],
                        acc.at[pl.ds(base, RPT)])

      plsc.subcore_barrier()

      def gissue(r, hoff):
        for b in range(RB):
          pltpu.async_copy(src_hbm.at[p].at[row_v.at[r * RB + b]],
                           buf.at[pl.ds(hoff + b * K, K)], gsem)

      def gwait(r, hoff):
        for b in range(RB):
          pltpu.make_async_copy(src_hbm.at[p].at[row_v.at[r * RB + b]],
                                buf.at[pl.ds(hoff + b * K, K)], gsem).wait()

      def sissue(r, hoff):
        for b in range(RB):
          pltpu.async_copy(buf.at[pl.ds(hoff + b * K, K)],
                           acc.at[col_v.at[r * RB + b]], ssem, add=True)

      def swait(r, hoff):
        for b in range(RB):
          pltpu.make_async_copy(buf.at[pl.ds(hoff + b * K, K)],
                                acc.at[col_v.at[r * RB + b]], ssem).wait()

      gissue(0, 0)

      def round_body(r, carry):
        hoff = (r % 2) * CH
        ooff = CH - hoff
        gwait(r, hoff)

        @pl.when(r >= 1)
        def _():
          swait(r - 1, ooff)

        @pl.when(r + 1 < NROUND)
        def _():
          gissue(r + 1, ooff)

        sissue(r, hoff)
        return carry

      lax.fori_loop(0, NROUND, round_body, 0)
      swait(NROUND - 1, ((NROUND - 1) % 2) * CH)
      plsc.subcore_barrier()
      pltpu.sync_copy(acc.at[pl.ds(base, RPT)],
                      out_hbm.at[p].at[c].at[pl.ds(base, RPT)])

  return agg_kernel


_deg_kernel = _make_deg_kernel()
_agg64x2 = _make_agg_kernel(64, 2)
_agg48 = _make_agg_kernel(48, 1)


_RBLK = 640
_NBLK = NP // _RBLK


def _prescale_body(deg_ref, x_ref, xs_ref, dinv_ref):
  deg = deg_ref[:, 0:1] + deg_ref[:, 1:2] + 1.0
  dinv = lax.rsqrt(deg)
  dinv_ref[...] = dinv
  xs_ref[0] = x_ref[:, :64] * dinv
  xs_ref[1] = x_ref[:, 64:] * dinv


def _tc_prescale(deg_t, x_pad):
  return pl.pallas_call(
      _prescale_body,
      grid=(_NBLK,),
      in_specs=[
          pl.BlockSpec((_RBLK, NC), lambda i: (i, 0)),
          pl.BlockSpec((_RBLK, 128), lambda i: (i, 0)),
      ],
      out_specs=[
          pl.BlockSpec((2, _RBLK, 64), lambda i: (0, i, 0)),
          pl.BlockSpec((_RBLK, 1), lambda i: (i, 0)),
      ],
      out_shape=[
          jax.ShapeDtypeStruct((2, NP, 64), jnp.float32),
          jax.ShapeDtypeStruct((NP, 1), jnp.float32),
      ],
  )(deg_t, x_pad)


def _mm_body(p_ref, dinv_ref, w1_ref, b1_ref, w2_ref, ys_ref):
  dinv = dinv_ref[...]
  a = jnp.concatenate([p_ref[0, 0] + p_ref[0, 1],
                       p_ref[1, 0] + p_ref[1, 1]], axis=1) * dinv
  h = jnp.dot(a, w1_ref[...], preferred_element_type=jnp.float32)
  h = jnp.maximum(h + b1_ref[...], 0.0)
  y = jnp.dot(h, w2_ref[...], preferred_element_type=jnp.float32)
  ys_ref[...] = y * dinv


def _tc_mm(p, dinv, w1, b1, w2p):
  return pl.pallas_call(
      _mm_body,
      grid=(_NBLK,),
      in_specs=[
          pl.BlockSpec((2, NC, _RBLK, 64), lambda i: (0, 0, i, 0)),
          pl.BlockSpec((_RBLK, 1), lambda i: (i, 0)),
          pl.BlockSpec((128, 2048), lambda i: (0, 0)),
          pl.BlockSpec((1, 2048), lambda i: (0, 0)),
          pl.BlockSpec((2048, 48), lambda i: (0, 0)),
      ],
      out_specs=pl.BlockSpec((_RBLK, 48), lambda i: (i, 0)),
      out_shape=jax.ShapeDtypeStruct((NP, 48), jnp.float32),
  )(p, dinv, w1, b1, w2p)


def _final_body(q_ref, dinv_ref, b2_ref, out_ref):
  out_ref[...] = (q_ref[0] + q_ref[1]) * dinv_ref[...] + b2_ref[...]


def _tc_final(q, dinv, b2p):
  return pl.pallas_call(
      _final_body,
      grid=(_NBLK,),
      in_specs=[
          pl.BlockSpec((NC, _RBLK, 48), lambda i: (0, i, 0)),
          pl.BlockSpec((_RBLK, 1), lambda i: (i, 0)),
          pl.BlockSpec((1, 48), lambda i: (0, 0)),
      ],
      out_specs=pl.BlockSpec((_RBLK, 48), lambda i: (i, 0)),
      out_shape=jax.ShapeDtypeStruct((NP, 48), jnp.float32),
  )(q, dinv, b2p)


def kernel(x, edge_index, W1, b1, W2, b2):
  ei = edge_index.astype(jnp.int32)
  row2 = ei[0].reshape(NW, NCHUNK, K)
  col2 = ei[1].reshape(NW, NCHUNK, K)
  x_pad = jnp.pad(x, ((0, NP - N), (0, 0)))
  w2p = jnp.pad(W2, ((0, 0), (0, 48 - W2.shape[1])))
  b1r = b1.reshape(1, 2048)
  b2p = jnp.pad(b2, (0, 48 - b2.shape[0])).reshape(1, 48)
  ones_k = jnp.ones((K,), jnp.float32)
  z1 = jnp.zeros((NP,), jnp.float32)
  z64 = jnp.zeros((NP, 64), jnp.float32)
  z48 = jnp.zeros((NP, 48), jnp.float32)

  degp = _deg_kernel(col2, ones_k, z1)
  deg_t = degp.T
  xs2, dinv = _tc_prescale(deg_t, x_pad)
  p = _agg64x2(row2, col2, xs2, z64)
  ys = _tc_mm(p, dinv, W1, b1r, w2p)
  q = _agg48(row2, col2, ys[None], z48)
  outp = _tc_final(q[0], dinv, b2p)
  return outp[:N, :40]
